# Initial kernel scaffold; baseline (speedup 1.0000x reference)
import functools

import jax
import jax.numpy as jnp
from jax import lax
from jax.experimental import pallas as pl
from jax.experimental.pallas import tpu as pltpu
from jax.experimental.pallas import tpu_sc as plsc

N = 10000
E = 320000
D_EDGE = 16
D_NODE = 128
OUT = 128

NC = 2
NS = 16
NW = NC * NS

SUB = 128
K = 20
CHUNK = K * SUB
T = 4
PER_TILE = T * CHUNK
E_PAD = NW * PER_TILE
IDX_ROWS = E_PAD // SUB

N_PAD = 10016
ZROWS = N_PAD // NS


def _sc_scatter_body(edge_hbm, recv_hbm, send_hbm, zero_hbm,
                     pin_hbm, pout_hbm,
                     rows_v, idxr_v, idxs_v, agg_in, agg_out):
    c = lax.axis_index("c")
    s = lax.axis_index("s")
    wid = c * NS + s

    pltpu.sync_copy(zero_hbm, agg_in.at[pl.ds(s * ZROWS, ZROWS)])
    pltpu.sync_copy(zero_hbm, agg_out.at[pl.ds(s * ZROWS, ZROWS)])
    plsc.subcore_barrier()

    def chunk_body(t, _):
        b = wid * PER_TILE + t * CHUNK
        row_base = jnp.minimum(b, E - CHUNK)
        pltpu.sync_copy(edge_hbm.at[pl.ds(row_base, CHUNK)], rows_v)
        ib = wid * (T * K) + t * K
        pltpu.sync_copy(recv_hbm.at[pl.ds(ib, K)], idxr_v)
        pltpu.sync_copy(send_hbm.at[pl.ds(ib, K)], idxs_v)

        def sub_body(j, _):
            src = rows_v.at[pl.ds(j * SUB, SUB)]
            pltpu.sync_copy(src, agg_in.at[idxr_v.at[j]], add=True)
            pltpu.sync_copy(src, agg_out.at[idxs_v.at[j]], add=True)
            return 0

        lax.fori_loop(0, K, sub_body, 0)
        return 0

    lax.fori_loop(0, T, chunk_body, 0)
    plsc.subcore_barrier()

    rows_out = N // NS
    sl = pl.ds(s * rows_out, rows_out)
    pltpu.sync_copy(agg_in.at[sl], pin_hbm.at[c].at[sl])
    pltpu.sync_copy(agg_out.at[sl], pout_hbm.at[c].at[sl])


_sc_scatter = pl.kernel(
    _sc_scatter_body,
    out_type=(
        jax.ShapeDtypeStruct((NC, N, D_EDGE), jnp.float32),
        jax.ShapeDtypeStruct((NC, N, D_EDGE), jnp.float32),
    ),
    mesh=plsc.VectorSubcoreMesh(core_axis_name="c", subcore_axis_name="s"),
    scratch_types=[
        pltpu.VMEM((CHUNK, D_EDGE), jnp.float32),
        pltpu.VMEM((K, SUB), jnp.int32),
        pltpu.VMEM((K, SUB), jnp.int32),
        pltpu.VMEM_SHARED((N_PAD, D_EDGE), jnp.float32),
        pltpu.VMEM_SHARED((N_PAD, D_EDGE), jnp.float32),
    ],
)


def _tc_linear_body(nf, pin, pout, wnt, wit, wot, b, out):
    agg_i = pin[0] + pin[1]
    agg_o = pout[0] + pout[1]
    acc = jnp.dot(nf[...], wnt[...],
                  preferred_element_type=jnp.float32, precision="highest")
    acc = acc + jnp.dot(agg_i, wit[...],
                        preferred_element_type=jnp.float32, precision="highest")
    acc = acc + jnp.dot(agg_o, wot[...],
                        preferred_element_type=jnp.float32, precision="highest")
    out[...] = acc + b[...]


_ROWS_BLK = 2000


def _tc_linear(nf, pin, pout, wnt, wit, wot, bias2d):
    grid = (N // _ROWS_BLK,)
    return pl.pallas_call(
        _tc_linear_body,
        grid=grid,
        in_specs=[
            pl.BlockSpec((_ROWS_BLK, D_NODE), lambda i: (i, 0)),
            pl.BlockSpec((NC, _ROWS_BLK, D_EDGE), lambda i: (0, i, 0)),
            pl.BlockSpec((NC, _ROWS_BLK, D_EDGE), lambda i: (0, i, 0)),
            pl.BlockSpec((D_NODE, OUT), lambda i: (0, 0)),
            pl.BlockSpec((D_EDGE, OUT), lambda i: (0, 0)),
            pl.BlockSpec((D_EDGE, OUT), lambda i: (0, 0)),
            pl.BlockSpec((1, OUT), lambda i: (0, 0)),
        ],
        out_specs=pl.BlockSpec((_ROWS_BLK, OUT), lambda i: (i, 0)),
        out_shape=jax.ShapeDtypeStruct((N, OUT), jnp.float32),
    )(nf, pin, pout, wnt, wit, wot, bias2d)


def kernel(node_features, edge_features, senders, receivers,
           W_node, W_incoming, W_outgoing, bias):
    pad = jnp.full((E_PAD - E,), N, dtype=jnp.int32)
    recv2d = jnp.concatenate([receivers, pad]).reshape(IDX_ROWS, SUB)
    send2d = jnp.concatenate([senders, pad]).reshape(IDX_ROWS, SUB)
    zeros = jnp.zeros((ZROWS, D_EDGE), jnp.float32)
    pin, pout = _sc_scatter(edge_features, recv2d, send2d, zeros)
    return _tc_linear(node_features, pin, pout,
                      W_node.T, W_incoming.T, W_outgoing.T,
                      bias.reshape(1, OUT))


# speedup vs baseline: 7.3706x; 7.3706x over previous
import functools

import jax
import jax.numpy as jnp
from jax import lax
from jax.experimental import pallas as pl
from jax.experimental.pallas import tpu as pltpu
from jax.experimental.pallas import tpu_sc as plsc

N = 10000
E = 320000
D_EDGE = 16
D_NODE = 128
OUT = 128

NC = 2
NS = 16
NW = NC * NS

SUB = 128
K = 16
CHUNK = K * SUB
T = 5
PER_TILE = T * CHUNK
E_PAD = NW * PER_TILE
IDX_ROWS = E_PAD // SUB

N_PAD = 10112
ZROWS = N_PAD // NS


def _sc_scatter_body(edge_hbm, recv_hbm, send_hbm, zero_hbm,
                     pin_hbm, pout_hbm,
                     rows_v, idxr_v, idxs_v, agg_in, agg_out):
    c = lax.axis_index("c")
    s = lax.axis_index("s")
    wid = c * NS + s

    pltpu.sync_copy(zero_hbm, agg_in.at[pl.ds(s * ZROWS, ZROWS)])
    pltpu.sync_copy(zero_hbm, agg_out.at[pl.ds(s * ZROWS, ZROWS)])
    plsc.subcore_barrier()

    def chunk_body(t, _):
        b = wid * PER_TILE + t * CHUNK
        row_base = jnp.minimum(b, E - CHUNK)
        pltpu.sync_copy(edge_hbm.at[pl.ds(row_base, CHUNK)], rows_v)
        ib = wid * (T * K) + t * K
        pltpu.sync_copy(recv_hbm.at[pl.ds(ib, K)], idxr_v)
        pltpu.sync_copy(send_hbm.at[pl.ds(ib, K)], idxs_v)

        def sub_body(j, _):
            src = rows_v.at[pl.ds(j * SUB, SUB)]
            pltpu.sync_copy(src, agg_in.at[idxr_v.at[j]], add=True)
            pltpu.sync_copy(src, agg_out.at[idxs_v.at[j]], add=True)
            return 0

        lax.fori_loop(0, K, sub_body, 0)
        return 0

    lax.fori_loop(0, T, chunk_body, 0)
    plsc.subcore_barrier()

    @pl.when(s < NS - 1)
    def _():
        sl = pl.ds(s * ZROWS, ZROWS)
        pltpu.sync_copy(agg_in.at[sl], pin_hbm.at[c].at[sl])
        pltpu.sync_copy(agg_out.at[sl], pout_hbm.at[c].at[sl])

    @pl.when(s == NS - 1)
    def _():
        tail = N - (NS - 1) * ZROWS
        sl = pl.ds((NS - 1) * ZROWS, tail)
        pltpu.sync_copy(agg_in.at[sl], pin_hbm.at[c].at[sl])
        pltpu.sync_copy(agg_out.at[sl], pout_hbm.at[c].at[sl])


@functools.cache
def _sc_scatter():
  return pl.kernel(
    _sc_scatter_body,
    out_type=(
        jax.ShapeDtypeStruct((NC, N, D_EDGE), jnp.float32),
        jax.ShapeDtypeStruct((NC, N, D_EDGE), jnp.float32),
    ),
    mesh=plsc.VectorSubcoreMesh(core_axis_name="c", subcore_axis_name="s",
                                num_cores=NC, num_subcores=NS),
    compiler_params=pltpu.CompilerParams(use_tc_tiling_on_sc=False),
    scratch_types=[
        pltpu.VMEM((CHUNK, D_EDGE), jnp.float32),
        pltpu.VMEM((K, SUB), jnp.int32),
        pltpu.VMEM((K, SUB), jnp.int32),
        pltpu.VMEM_SHARED((N_PAD, D_EDGE), jnp.float32),
        pltpu.VMEM_SHARED((N_PAD, D_EDGE), jnp.float32),
    ],
  )


def _tc_linear_body(nf, pin, pout, wnt, wit, wot, b, out):
    agg_i = pin[0] + pin[1]
    agg_o = pout[0] + pout[1]
    acc = jnp.dot(nf[...], wnt[...],
                  preferred_element_type=jnp.float32, precision="highest")
    acc = acc + jnp.dot(agg_i, wit[...],
                        preferred_element_type=jnp.float32, precision="highest")
    acc = acc + jnp.dot(agg_o, wot[...],
                        preferred_element_type=jnp.float32, precision="highest")
    out[...] = acc + b[...]


_ROWS_BLK = 2000


def _tc_linear(nf, pin, pout, wnt, wit, wot, bias2d):
    grid = (N // _ROWS_BLK,)
    return pl.pallas_call(
        _tc_linear_body,
        grid=grid,
        in_specs=[
            pl.BlockSpec((_ROWS_BLK, D_NODE), lambda i: (i, 0)),
            pl.BlockSpec((NC, _ROWS_BLK, D_EDGE), lambda i: (0, i, 0)),
            pl.BlockSpec((NC, _ROWS_BLK, D_EDGE), lambda i: (0, i, 0)),
            pl.BlockSpec((D_NODE, OUT), lambda i: (0, 0)),
            pl.BlockSpec((D_EDGE, OUT), lambda i: (0, 0)),
            pl.BlockSpec((D_EDGE, OUT), lambda i: (0, 0)),
            pl.BlockSpec((1, OUT), lambda i: (0, 0)),
        ],
        out_specs=pl.BlockSpec((_ROWS_BLK, OUT), lambda i: (i, 0)),
        out_shape=jax.ShapeDtypeStruct((N, OUT), jnp.float32),
    )(nf, pin, pout, wnt, wit, wot, bias2d)


B_PART = (E // CHUNK) * CHUNK
CLAMP = E - CHUNK
SHIFT = B_PART - CLAMP
TAIL = E - B_PART


def _pad_idx(ix):
    return jnp.concatenate([
        ix[:B_PART],
        jnp.full((SHIFT,), N, dtype=jnp.int32),
        ix[B_PART:],
        jnp.full((E_PAD - B_PART - SHIFT - TAIL,), N, dtype=jnp.int32),
    ]).reshape(IDX_ROWS, SUB)


def kernel(node_features, edge_features, senders, receivers,
           W_node, W_incoming, W_outgoing, bias):
    recv2d = _pad_idx(receivers)
    send2d = _pad_idx(senders)
    zeros = jnp.zeros((ZROWS, D_EDGE), jnp.float32)
    pin, pout = _sc_scatter()(edge_features, recv2d, send2d, zeros)
    return _tc_linear(node_features, pin, pout,
                      W_node.T, W_incoming.T, W_outgoing.T,
                      bias.reshape(1, OUT))


# speedup vs baseline: 7.6265x; 1.0347x over previous
import functools

import jax
import jax.numpy as jnp
from jax import lax
from jax.experimental import pallas as pl
from jax.experimental.pallas import tpu as pltpu
from jax.experimental.pallas import tpu_sc as plsc

N = 10000
E = 320000
D_EDGE = 16
D_NODE = 128
OUT = 128

NC = 2
NS = 16
NW = NC * NS

SUB = 128
K = 16
CHUNK = K * SUB
T = 5
PER_TILE = T * CHUNK
E_PAD = NW * PER_TILE
IDX_ROWS = E_PAD // SUB

N_PAD = 10112
ZROWS = N_PAD // NS


ROWS2D = CHUNK // 8


def _sc_scatter_body(edge_hbm, recv_hbm, send_hbm, zero_hbm,
                     pin_hbm, pout_hbm,
                     rows_a, rows_b, idxr_a, idxr_b, idxs_a, idxs_b,
                     agg_in, agg_out,
                     sem_a, sem_b, sem_sc):
    c = lax.axis_index("c")
    s = lax.axis_index("s")
    wid = c * NS + s

    @pl.when(s == 0)
    def _():
        pltpu.sync_copy(zero_hbm, agg_in)

    @pl.when(s == 1)
    def _():
        pltpu.sync_copy(zero_hbm, agg_out)

    plsc.subcore_barrier()

    def start_stage(t, rows_v, idxr_v, idxs_v, sem):
        b = wid * PER_TILE + t * CHUNK
        row_base = jnp.minimum(b, E - CHUNK)
        pltpu.async_copy(edge_hbm.at[pl.ds(row_base, CHUNK)], rows_v, sem)
        ib = wid * (T * K) + t * K
        pltpu.async_copy(recv_hbm.at[pl.ds(ib, K)], idxr_v, sem)
        pltpu.async_copy(send_hbm.at[pl.ds(ib, K)], idxs_v, sem)

    def wait_stage(rows_v, idxr_v, idxs_v, sem):
        pltpu.make_async_copy(edge_hbm.at[pl.ds(0, CHUNK)], rows_v, sem).wait()
        pltpu.make_async_copy(recv_hbm.at[pl.ds(0, K)], idxr_v, sem).wait()
        pltpu.make_async_copy(send_hbm.at[pl.ds(0, K)], idxs_v, sem).wait()

    def do_chunk(t, rows_v, idxr_v, idxs_v, sem,
                 rows_n, idxr_n, idxs_n, sem_n):
        wait_stage(rows_v, idxr_v, idxs_v, sem)

        @pl.when(t + 1 < T)
        def _():
            start_stage(t + 1, rows_n, idxr_n, idxs_n, sem_n)

        def sub_body(j, _):
            src = rows_v.at[pl.ds(j * SUB, SUB)]
            pltpu.async_copy(src, agg_in.at[idxr_v.at[j]], sem_sc, add=True)
            pltpu.async_copy(src, agg_out.at[idxs_v.at[j]], sem_sc, add=True)
            return 0

        lax.fori_loop(0, K, sub_body, 0)
        pltpu.make_async_copy(edge_hbm.at[pl.ds(0, CHUNK)], rows_v, sem_sc).wait()
        pltpu.make_async_copy(edge_hbm.at[pl.ds(0, CHUNK)], rows_v, sem_sc).wait()

    start_stage(0, rows_a, idxr_a, idxs_a, sem_a)

    def outer(t, _):
        @pl.when(t % 2 == 0)
        def _():
            do_chunk(t, rows_a, idxr_a, idxs_a, sem_a,
                     rows_b, idxr_b, idxs_b, sem_b)

        @pl.when(t % 2 == 1)
        def _():
            do_chunk(t, rows_b, idxr_b, idxs_b, sem_b,
                     rows_a, idxr_a, idxs_a, sem_a)

        return 0

    lax.fori_loop(0, T, outer, 0)
    plsc.subcore_barrier()

    @pl.when(s < NS - 1)
    def _():
        sl = pl.ds(s * ZROWS, ZROWS)
        pltpu.sync_copy(agg_in.at[sl], pin_hbm.at[c].at[sl])
        pltpu.sync_copy(agg_out.at[sl], pout_hbm.at[c].at[sl])

    @pl.when(s == NS - 1)
    def _():
        tail = N - (NS - 1) * ZROWS
        sl = pl.ds((NS - 1) * ZROWS, tail)
        pltpu.sync_copy(agg_in.at[sl], pin_hbm.at[c].at[sl])
        pltpu.sync_copy(agg_out.at[sl], pout_hbm.at[c].at[sl])


@functools.cache
def _sc_scatter():
  return pl.kernel(
    _sc_scatter_body,
    out_type=(
        jax.ShapeDtypeStruct((NC, N, D_EDGE), jnp.float32),
        jax.ShapeDtypeStruct((NC, N, D_EDGE), jnp.float32),
    ),
    mesh=plsc.VectorSubcoreMesh(core_axis_name="c", subcore_axis_name="s",
                                num_cores=NC, num_subcores=NS),
    compiler_params=pltpu.CompilerParams(use_tc_tiling_on_sc=False),
    scratch_types=[
        pltpu.VMEM((CHUNK, D_EDGE), jnp.float32),
        pltpu.VMEM((CHUNK, D_EDGE), jnp.float32),
        pltpu.VMEM((K, SUB), jnp.int32),
        pltpu.VMEM((K, SUB), jnp.int32),
        pltpu.VMEM((K, SUB), jnp.int32),
        pltpu.VMEM((K, SUB), jnp.int32),
        pltpu.VMEM_SHARED((N_PAD, D_EDGE), jnp.float32),
        pltpu.VMEM_SHARED((N_PAD, D_EDGE), jnp.float32),
        pltpu.SemaphoreType.DMA,
        pltpu.SemaphoreType.DMA,
        pltpu.SemaphoreType.DMA,
    ],
  )


def _tc_linear_body(nf, pin, pout, wnt, wit, wot, b, out):
    agg_i = pin[0] + pin[1]
    agg_o = pout[0] + pout[1]
    acc = jnp.dot(nf[...], wnt[...],
                  preferred_element_type=jnp.float32, precision="highest")
    acc = acc + jnp.dot(agg_i, wit[...],
                        preferred_element_type=jnp.float32, precision="highest")
    acc = acc + jnp.dot(agg_o, wot[...],
                        preferred_element_type=jnp.float32, precision="highest")
    out[...] = acc + b[...]


_ROWS_BLK = 2000


def _tc_linear(nf, pin, pout, wnt, wit, wot, bias2d):
    grid = (N // _ROWS_BLK,)
    return pl.pallas_call(
        _tc_linear_body,
        grid=grid,
        in_specs=[
            pl.BlockSpec((_ROWS_BLK, D_NODE), lambda i: (i, 0)),
            pl.BlockSpec((NC, _ROWS_BLK, D_EDGE), lambda i: (0, i, 0)),
            pl.BlockSpec((NC, _ROWS_BLK, D_EDGE), lambda i: (0, i, 0)),
            pl.BlockSpec((D_NODE, OUT), lambda i: (0, 0)),
            pl.BlockSpec((D_EDGE, OUT), lambda i: (0, 0)),
            pl.BlockSpec((D_EDGE, OUT), lambda i: (0, 0)),
            pl.BlockSpec((1, OUT), lambda i: (0, 0)),
        ],
        out_specs=pl.BlockSpec((_ROWS_BLK, OUT), lambda i: (i, 0)),
        out_shape=jax.ShapeDtypeStruct((N, OUT), jnp.float32),
    )(nf, pin, pout, wnt, wit, wot, bias2d)


B_PART = (E // CHUNK) * CHUNK
CLAMP = E - CHUNK
SHIFT = B_PART - CLAMP
TAIL = E - B_PART


def _pad_idx(ix):
    return jnp.concatenate([
        ix[:B_PART],
        jnp.full((SHIFT,), N, dtype=jnp.int32),
        ix[B_PART:],
        jnp.full((E_PAD - B_PART - SHIFT - TAIL,), N, dtype=jnp.int32),
    ]).reshape(IDX_ROWS, SUB)


def kernel(node_features, edge_features, senders, receivers,
           W_node, W_incoming, W_outgoing, bias):
    recv2d = _pad_idx(receivers)
    send2d = _pad_idx(senders)
    zeros = jnp.zeros((N_PAD, D_EDGE), jnp.float32)
    pin, pout = _sc_scatter()(edge_features, recv2d, send2d, zeros)
    return _tc_linear(node_features, pin, pout,
                      W_node.T, W_incoming.T, W_outgoing.T,
                      bias.reshape(1, OUT))


# speedup vs baseline: 7.6828x; 1.0074x over previous
import functools

import jax
import jax.numpy as jnp
from jax import lax
from jax.experimental import pallas as pl
from jax.experimental.pallas import tpu as pltpu
from jax.experimental.pallas import tpu_sc as plsc

N = 10000
E = 320000
D_EDGE = 16
D_NODE = 128
OUT = 128

NC = 2
NS = 16
NW = NC * NS

SUB = 128
K = 16
CHUNK = K * SUB
T = 5
PER_TILE = T * CHUNK
E_PAD = NW * PER_TILE
IDX_ROWS = E_PAD // SUB

N_PAD = 10112
ZROWS = N_PAD // NS


ROWS2D = CHUNK // 8


def _sc_scatter_body(edge_hbm, recv_hbm, send_hbm, zero_hbm,
                     pin_hbm, pout_hbm,
                     rows_a, rows_b, idxr_a, idxr_b, idxs_a, idxs_b,
                     agg_in, agg_out,
                     sem_a, sem_b, sem_sc):
    c = lax.axis_index("c")
    s = lax.axis_index("s")
    wid = c * NS + s

    @pl.when(s == 0)
    def _():
        pltpu.sync_copy(zero_hbm, agg_in)

    @pl.when(s == 1)
    def _():
        pltpu.sync_copy(zero_hbm, agg_out)

    plsc.subcore_barrier()

    def start_stage(t, rows_v, idxr_v, idxs_v, sem):
        b = wid * PER_TILE + t * CHUNK
        row_base = jnp.minimum(b, E - CHUNK)
        pltpu.async_copy(edge_hbm.at[pl.ds(row_base, CHUNK)], rows_v, sem)
        pltpu.async_copy(recv_hbm.at[pl.ds(b, CHUNK)], idxr_v, sem)
        pltpu.async_copy(send_hbm.at[pl.ds(b, CHUNK)], idxs_v, sem)

    def wait_stage(rows_v, idxr_v, idxs_v, sem):
        pltpu.make_async_copy(edge_hbm.at[pl.ds(0, CHUNK)], rows_v, sem).wait()
        pltpu.make_async_copy(recv_hbm.at[pl.ds(0, CHUNK)], idxr_v, sem).wait()
        pltpu.make_async_copy(send_hbm.at[pl.ds(0, CHUNK)], idxs_v, sem).wait()

    def do_chunk(t, rows_v, idxr_v, idxs_v, sem,
                 rows_n, idxr_n, idxs_n, sem_n):
        wait_stage(rows_v, idxr_v, idxs_v, sem)

        @pl.when(t + 1 < T)
        def _():
            start_stage(t + 1, rows_n, idxr_n, idxs_n, sem_n)

        def sub_body(j, _):
            src = rows_v.at[pl.ds(j * SUB, SUB)]
            ix = pl.ds(j * SUB, SUB)
            pltpu.async_copy(src, agg_in.at[idxr_v.at[ix]], sem_sc, add=True)
            pltpu.async_copy(src, agg_out.at[idxs_v.at[ix]], sem_sc, add=True)
            return 0

        lax.fori_loop(0, K, sub_body, 0)
        pltpu.make_async_copy(edge_hbm.at[pl.ds(0, CHUNK)], rows_v, sem_sc).wait()
        pltpu.make_async_copy(edge_hbm.at[pl.ds(0, CHUNK)], rows_v, sem_sc).wait()

    start_stage(0, rows_a, idxr_a, idxs_a, sem_a)

    def outer(t, _):
        @pl.when(t % 2 == 0)
        def _():
            do_chunk(t, rows_a, idxr_a, idxs_a, sem_a,
                     rows_b, idxr_b, idxs_b, sem_b)

        @pl.when(t % 2 == 1)
        def _():
            do_chunk(t, rows_b, idxr_b, idxs_b, sem_b,
                     rows_a, idxr_a, idxs_a, sem_a)

        return 0

    lax.fori_loop(0, T, outer, 0)
    plsc.subcore_barrier()

    @pl.when(s < NS - 1)
    def _():
        sl = pl.ds(s * ZROWS, ZROWS)
        pltpu.sync_copy(agg_in.at[sl], pin_hbm.at[c].at[sl])
        pltpu.sync_copy(agg_out.at[sl], pout_hbm.at[c].at[sl])

    @pl.when(s == NS - 1)
    def _():
        tail = N - (NS - 1) * ZROWS
        sl = pl.ds((NS - 1) * ZROWS, tail)
        pltpu.sync_copy(agg_in.at[sl], pin_hbm.at[c].at[sl])
        pltpu.sync_copy(agg_out.at[sl], pout_hbm.at[c].at[sl])


@functools.cache
def _sc_scatter():
  return pl.kernel(
    _sc_scatter_body,
    out_type=(
        jax.ShapeDtypeStruct((NC, N, D_EDGE), jnp.float32),
        jax.ShapeDtypeStruct((NC, N, D_EDGE), jnp.float32),
    ),
    mesh=plsc.VectorSubcoreMesh(core_axis_name="c", subcore_axis_name="s",
                                num_cores=NC, num_subcores=NS),
    compiler_params=pltpu.CompilerParams(use_tc_tiling_on_sc=False),
    scratch_types=[
        pltpu.VMEM((CHUNK, D_EDGE), jnp.float32),
        pltpu.VMEM((CHUNK, D_EDGE), jnp.float32),
        pltpu.VMEM((CHUNK,), jnp.int32),
        pltpu.VMEM((CHUNK,), jnp.int32),
        pltpu.VMEM((CHUNK,), jnp.int32),
        pltpu.VMEM((CHUNK,), jnp.int32),
        pltpu.VMEM_SHARED((N_PAD, D_EDGE), jnp.float32),
        pltpu.VMEM_SHARED((N_PAD, D_EDGE), jnp.float32),
        pltpu.SemaphoreType.DMA,
        pltpu.SemaphoreType.DMA,
        pltpu.SemaphoreType.DMA,
    ],
  )


def _tc_linear_body(nf, pin, pout, wnt, wit, wot, b, out):
    agg_i = pin[0] + pin[1]
    agg_o = pout[0] + pout[1]
    acc = jnp.dot(nf[...], wnt[...],
                  preferred_element_type=jnp.float32, precision="highest")
    acc = acc + jnp.dot(agg_i, wit[...],
                        preferred_element_type=jnp.float32, precision="highest")
    acc = acc + jnp.dot(agg_o, wot[...],
                        preferred_element_type=jnp.float32, precision="highest")
    out[...] = acc + b[...]


_ROWS_BLK = 2000


def _tc_linear(nf, pin, pout, wnt, wit, wot, bias2d):
    grid = (N // _ROWS_BLK,)
    return pl.pallas_call(
        _tc_linear_body,
        grid=grid,
        in_specs=[
            pl.BlockSpec((_ROWS_BLK, D_NODE), lambda i: (i, 0)),
            pl.BlockSpec((NC, _ROWS_BLK, D_EDGE), lambda i: (0, i, 0)),
            pl.BlockSpec((NC, _ROWS_BLK, D_EDGE), lambda i: (0, i, 0)),
            pl.BlockSpec((D_NODE, OUT), lambda i: (0, 0)),
            pl.BlockSpec((D_EDGE, OUT), lambda i: (0, 0)),
            pl.BlockSpec((D_EDGE, OUT), lambda i: (0, 0)),
            pl.BlockSpec((1, OUT), lambda i: (0, 0)),
        ],
        out_specs=pl.BlockSpec((_ROWS_BLK, OUT), lambda i: (i, 0)),
        out_shape=jax.ShapeDtypeStruct((N, OUT), jnp.float32),
    )(nf, pin, pout, wnt, wit, wot, bias2d)


B_PART = (E // CHUNK) * CHUNK
CLAMP = E - CHUNK
SHIFT = B_PART - CLAMP
TAIL = E - B_PART


def _pad_idx(ix):
    return jnp.concatenate([
        ix[:B_PART],
        jnp.full((SHIFT,), N, dtype=jnp.int32),
        ix[B_PART:],
        jnp.full((E_PAD - B_PART - SHIFT - TAIL,), N, dtype=jnp.int32),
    ])


def kernel(node_features, edge_features, senders, receivers,
           W_node, W_incoming, W_outgoing, bias):
    recv2d = _pad_idx(receivers)
    send2d = _pad_idx(senders)
    zeros = jnp.zeros((N_PAD, D_EDGE), jnp.float32)
    pin, pout = _sc_scatter()(edge_features, recv2d, send2d, zeros)
    return _tc_linear(node_features, pin, pout,
                      W_node.T, W_incoming.T, W_outgoing.T,
                      bias.reshape(1, OUT))


# speedup vs baseline: 7.9604x; 1.0361x over previous
import functools

import jax
import jax.numpy as jnp
from jax import lax
from jax.experimental import pallas as pl
from jax.experimental.pallas import tpu as pltpu
from jax.experimental.pallas import tpu_sc as plsc

N = 10000
E = 320000
D_EDGE = 16
D_NODE = 128
OUT = 128

NC = 2
NS = 16
NW = NC * NS

SUB = 128
K = 16
CHUNK = K * SUB
T = 5
PER_TILE = T * CHUNK
E_PAD = NW * PER_TILE
IDX_ROWS = E_PAD // SUB

N_PAD = 10112
ZROWS = N_PAD // NS


ROWS2D = CHUNK // 8


def _sc_scatter_body(edge_hbm, recv_hbm, send_hbm, zero_hbm,
                     pin_hbm, pout_hbm,
                     rows_a, rows_b, idxr_a, idxr_b, idxs_a, idxs_b,
                     agg_in, agg_out,
                     sem_a, sem_b, sem_sc):
    c = lax.axis_index("c")
    s = lax.axis_index("s")
    wid = c * NS + s

    @pl.when(s == 0)
    def _():
        pltpu.sync_copy(zero_hbm, agg_in)

    @pl.when(s == 1)
    def _():
        pltpu.sync_copy(zero_hbm, agg_out)

    plsc.subcore_barrier()

    def start_stage(t, rows_v, idxr_v, idxs_v, sem):
        b = wid * PER_TILE + t * CHUNK
        row_base = jnp.minimum(b, E - CHUNK)
        pltpu.async_copy(edge_hbm.at[pl.ds(row_base, CHUNK)], rows_v, sem)
        pltpu.async_copy(recv_hbm.at[pl.ds(b, CHUNK)], idxr_v, sem)
        pltpu.async_copy(send_hbm.at[pl.ds(b, CHUNK)], idxs_v, sem)

    def wait_stage(rows_v, idxr_v, idxs_v, sem):
        pltpu.make_async_copy(edge_hbm.at[pl.ds(0, CHUNK)], rows_v, sem).wait()
        pltpu.make_async_copy(recv_hbm.at[pl.ds(0, CHUNK)], idxr_v, sem).wait()
        pltpu.make_async_copy(send_hbm.at[pl.ds(0, CHUNK)], idxs_v, sem).wait()

    def do_chunk(t, rows_v, idxr_v, idxs_v, sem,
                 rows_n, idxr_n, idxs_n, sem_n):
        wait_stage(rows_v, idxr_v, idxs_v, sem)

        @pl.when(t + 1 < T)
        def _():
            start_stage(t + 1, rows_n, idxr_n, idxs_n, sem_n)

        def sub_body(j, _):
            src = rows_v.at[pl.ds(j * SUB, SUB)]
            ix = pl.ds(j * SUB, SUB)
            pltpu.async_copy(src, agg_in.at[idxr_v.at[ix]], sem_sc, add=True)
            pltpu.async_copy(src, agg_out.at[idxs_v.at[ix]], sem_sc, add=True)
            return 0

        lax.fori_loop(0, K, sub_body, 0)
        pltpu.make_async_copy(edge_hbm.at[pl.ds(0, CHUNK)], rows_v, sem_sc).wait()
        pltpu.make_async_copy(edge_hbm.at[pl.ds(0, CHUNK)], rows_v, sem_sc).wait()

    start_stage(0, rows_a, idxr_a, idxs_a, sem_a)

    def outer(t, _):
        @pl.when(t % 2 == 0)
        def _():
            do_chunk(t, rows_a, idxr_a, idxs_a, sem_a,
                     rows_b, idxr_b, idxs_b, sem_b)

        @pl.when(t % 2 == 1)
        def _():
            do_chunk(t, rows_b, idxr_b, idxs_b, sem_b,
                     rows_a, idxr_a, idxs_a, sem_a)

        return 0

    lax.fori_loop(0, T, outer, 0)
    plsc.subcore_barrier()

    @pl.when(s < NS - 1)
    def _():
        sl = pl.ds(s * ZROWS, ZROWS)
        pltpu.sync_copy(agg_in.at[sl], pin_hbm.at[c].at[sl])
        pltpu.sync_copy(agg_out.at[sl], pout_hbm.at[c].at[sl])

    @pl.when(s == NS - 1)
    def _():
        tail = N - (NS - 1) * ZROWS
        sl = pl.ds((NS - 1) * ZROWS, tail)
        pltpu.sync_copy(agg_in.at[sl], pin_hbm.at[c].at[sl])
        pltpu.sync_copy(agg_out.at[sl], pout_hbm.at[c].at[sl])


@functools.cache
def _sc_scatter():
  return pl.kernel(
    _sc_scatter_body,
    out_type=(
        jax.ShapeDtypeStruct((NC, N, D_EDGE), jnp.float32),
        jax.ShapeDtypeStruct((NC, N, D_EDGE), jnp.float32),
    ),
    mesh=plsc.VectorSubcoreMesh(core_axis_name="c", subcore_axis_name="s",
                                num_cores=NC, num_subcores=NS),
    compiler_params=pltpu.CompilerParams(use_tc_tiling_on_sc=False),
    scratch_types=[
        pltpu.VMEM((CHUNK, D_EDGE), jnp.float32),
        pltpu.VMEM((CHUNK, D_EDGE), jnp.float32),
        pltpu.VMEM((CHUNK,), jnp.int32),
        pltpu.VMEM((CHUNK,), jnp.int32),
        pltpu.VMEM((CHUNK,), jnp.int32),
        pltpu.VMEM((CHUNK,), jnp.int32),
        pltpu.VMEM_SHARED((N_PAD, D_EDGE), jnp.float32),
        pltpu.VMEM_SHARED((N_PAD, D_EDGE), jnp.float32),
        pltpu.SemaphoreType.DMA,
        pltpu.SemaphoreType.DMA,
        pltpu.SemaphoreType.DMA,
    ],
  )


_GBLK = N // 8


def _tc_linear_body(nf3, pinP, poutP, wnt, wci, wco, b, out3):
    pi = pinP[0] + pinP[1]
    po = poutP[0] + poutP[1]
    acc = lax.dot_general(nf3[...], wnt[...],
                          dimension_numbers=(((2,), (0,)), ((), ())),
                          preferred_element_type=jnp.float32,
                          precision="highest")
    acc = acc + lax.dot_general(pi, wci[...],
                                dimension_numbers=(((1,), (0,)), ((), ())),
                                preferred_element_type=jnp.float32,
                                precision="highest")
    acc = acc + lax.dot_general(po, wco[...],
                                dimension_numbers=(((1,), (0,)), ((), ())),
                                preferred_element_type=jnp.float32,
                                precision="highest")
    out3[...] = acc + b[...]


def _tc_linear(nf3, pinP, poutP, wnt, wci, wco, bias3d):
    pp = N * D_EDGE // 128
    return pl.pallas_call(
        _tc_linear_body,
        grid=(1,),
        in_specs=[
            pl.BlockSpec((_GBLK, 8, D_NODE), lambda i: (0, 0, 0)),
            pl.BlockSpec((NC, pp, 128), lambda i: (0, 0, 0)),
            pl.BlockSpec((NC, pp, 128), lambda i: (0, 0, 0)),
            pl.BlockSpec((D_NODE, OUT), lambda i: (0, 0)),
            pl.BlockSpec((128, 8, OUT), lambda i: (0, 0, 0)),
            pl.BlockSpec((128, 8, OUT), lambda i: (0, 0, 0)),
            pl.BlockSpec((1, 1, OUT), lambda i: (0, 0, 0)),
        ],
        out_specs=pl.BlockSpec((_GBLK, 8, OUT), lambda i: (0, 0, 0)),
        out_shape=jax.ShapeDtypeStruct((N // 8, 8, OUT), jnp.float32),
    )(nf3, pinP, poutP, wnt, wci, wco, bias3d)


def _lift_w(w):
    eye = jnp.eye(8, dtype=jnp.float32)
    wc = eye[:, None, :, None] * w.T[None, :, None, :]
    return wc.reshape(128, 8, OUT)


B_PART = (E // CHUNK) * CHUNK
CLAMP = E - CHUNK
SHIFT = B_PART - CLAMP
TAIL = E - B_PART


def _pad_idx(ix):
    return jnp.concatenate([
        ix[:B_PART],
        jnp.full((SHIFT,), N, dtype=jnp.int32),
        ix[B_PART:],
        jnp.full((E_PAD - B_PART - SHIFT - TAIL,), N, dtype=jnp.int32),
    ])


def kernel(node_features, edge_features, senders, receivers,
           W_node, W_incoming, W_outgoing, bias):
    recv2d = _pad_idx(receivers)
    send2d = _pad_idx(senders)
    zeros = jnp.zeros((N_PAD, D_EDGE), jnp.float32)
    pin, pout = _sc_scatter()(edge_features, recv2d, send2d, zeros)
    pp = N * D_EDGE // 128
    out3 = _tc_linear(node_features.reshape(N // 8, 8, D_NODE),
                      pin.reshape(NC, pp, 128),
                      pout.reshape(NC, pp, 128),
                      W_node.T, _lift_w(W_incoming), _lift_w(W_outgoing),
                      bias.reshape(1, 1, OUT))
    return out3.reshape(N, OUT)
